# baseline (speedup 1.0000x reference)
import functools

import jax
import jax.numpy as jnp
from jax import lax
from jax.experimental import pallas as pl
from jax.experimental.pallas import tpu as pltpu
from jax.experimental.pallas import tpu_sc as plsc

_LANES = 16


@functools.lru_cache(maxsize=None)
def _make_pool_kernel(B, CTX, D, NC, NS):
    NW = NC * NS
    b_per_w = B // NW
    n_idx = b_per_w * CTX
    inv_ctx = 1.0 / CTX

    mesh = plsc.VectorSubcoreMesh(
        core_axis_name="c", subcore_axis_name="s", num_cores=NC, num_subcores=NS
    )

    CH = 128
    n_chunks = n_idx // CH
    assert n_chunks * CH == n_idx

    @functools.partial(
        pl.kernel,
        mesh=mesh,
        out_type=jax.ShapeDtypeStruct((B, D), jnp.float32),
        scratch_types=[
            pltpu.VMEM((n_idx,), jnp.int32),
            pltpu.VMEM((n_idx, D), jnp.float32),
            pltpu.VMEM((b_per_w, D), jnp.float32),
            pltpu.SemaphoreType.DMA,
        ],
        compiler_params=pltpu.CompilerParams(use_tc_tiling_on_sc=False),
    )
    def pool(idx_hbm, table_hbm, out_hbm, idx_v, rows_v, pooled_v, sem):
        wid = lax.axis_index("s") * NC + lax.axis_index("c")
        pltpu.sync_copy(idx_hbm.at[pl.ds(wid * n_idx, n_idx)], idx_v)
        copies = [
            pltpu.async_copy(
                table_hbm.at[idx_v.at[pl.ds(c * CH, CH)]],
                rows_v.at[pl.ds(c * CH, CH)],
                sem,
            )
            for c in range(n_chunks)
        ]
        for cp in copies:
            cp.wait()

        def batch_body(i, carry):
            base = i * CTX

            def ctx_body(j, acc):
                a0, a1 = acc
                r = base + j
                a0 = a0 + rows_v[r, pl.ds(0, _LANES)]
                a1 = a1 + rows_v[r, pl.ds(_LANES, _LANES)]
                return (a0, a1)

            zero = jnp.zeros((_LANES,), jnp.float32)
            a0, a1 = lax.fori_loop(0, CTX, ctx_body, (zero, zero))
            pooled_v[i, pl.ds(0, _LANES)] = a0 * inv_ctx
            pooled_v[i, pl.ds(_LANES, _LANES)] = a1 * inv_ctx
            return carry

        lax.fori_loop(0, b_per_w, batch_body, 0)
        pltpu.sync_copy(pooled_v, out_hbm.at[pl.ds(wid * b_per_w, b_per_w)])

    return pool


def _proj_body(w_ref, p_ref, o_ref):
    o_ref[...] = jax.lax.dot_general(
        w_ref[...],
        p_ref[...],
        (((0,), (0,)), ((), ())),
        preferred_element_type=jnp.float32,
    )


@functools.lru_cache(maxsize=None)
def _make_proj_kernel(B, K, V, BN):
    grid = pl.cdiv(V, BN)
    return pl.pallas_call(
        _proj_body,
        grid=(grid,),
        in_specs=[
            pl.BlockSpec((K, BN), lambda i: (0, i)),
            pl.BlockSpec((K, B), lambda i: (0, 0)),
        ],
        out_specs=pl.BlockSpec((BN, B), lambda i: (i, 0)),
        out_shape=jax.ShapeDtypeStruct((V, B), jnp.float32),
    )


def kernel(context, emb_table, W, b):
    B, CTX = context.shape
    V, D = emb_table.shape
    NC, NS = 2, 16
    idx_flat = jnp.clip(context, 0, V - 1).reshape(-1)
    pooled = _make_pool_kernel(B, CTX, D, NC, NS)(idx_flat, emb_table)
    w_aug = jnp.concatenate([W, b[None, :]], axis=0)
    p_aug = jnp.concatenate(
        [pooled.T, jnp.ones((1, B), jnp.float32)], axis=0
    )
    out_t = _make_proj_kernel(B, D + 1, V, 2048)(w_aug, p_aug)
    return out_t.T


# speedup vs baseline: 1.0031x; 1.0031x over previous
import functools

import jax
import jax.numpy as jnp
from jax import lax
from jax.experimental import pallas as pl
from jax.experimental.pallas import tpu as pltpu
from jax.experimental.pallas import tpu_sc as plsc

_LANES = 16


@functools.lru_cache(maxsize=None)
def _make_pool_kernel(B, CTX, D, NC, NS):
    NW = NC * NS
    b_per_w = B // NW
    n_idx = b_per_w * CTX
    inv_ctx = 1.0 / CTX

    mesh = plsc.VectorSubcoreMesh(
        core_axis_name="c", subcore_axis_name="s", num_cores=NC, num_subcores=NS
    )

    CH = 128
    n_chunks = n_idx // CH
    assert n_chunks * CH == n_idx

    @functools.partial(
        pl.kernel,
        mesh=mesh,
        out_type=jax.ShapeDtypeStruct((B, D), jnp.float32),
        scratch_types=[
            pltpu.VMEM((n_idx,), jnp.int32),
            pltpu.VMEM((n_idx, D), jnp.float32),
            pltpu.VMEM((b_per_w, D), jnp.float32),
            pltpu.SemaphoreType.DMA,
        ],
        compiler_params=pltpu.CompilerParams(use_tc_tiling_on_sc=False),
    )
    def pool(idx_hbm, table_hbm, out_hbm, idx_v, rows_v, pooled_v, sem):
        wid = lax.axis_index("s") * NC + lax.axis_index("c")
        pltpu.sync_copy(idx_hbm.at[pl.ds(wid * n_idx, n_idx)], idx_v)
        copies = [
            pltpu.async_copy(
                table_hbm.at[idx_v.at[pl.ds(c * CH, CH)]],
                rows_v.at[pl.ds(c * CH, CH)],
                sem,
            )
            for c in range(n_chunks)
        ]
        for cp in copies:
            cp.wait()

        def batch_body(i, carry):
            base = i * CTX

            def ctx_body(j, acc):
                a0, a1 = acc
                r = base + j
                a0 = a0 + rows_v[r, pl.ds(0, _LANES)]
                a1 = a1 + rows_v[r, pl.ds(_LANES, _LANES)]
                return (a0, a1)

            zero = jnp.zeros((_LANES,), jnp.float32)
            a0, a1 = lax.fori_loop(0, CTX, ctx_body, (zero, zero))
            pooled_v[i, pl.ds(0, _LANES)] = a0 * inv_ctx
            pooled_v[i, pl.ds(_LANES, _LANES)] = a1 * inv_ctx
            return carry

        lax.fori_loop(0, b_per_w, batch_body, 0)
        pltpu.sync_copy(pooled_v, out_hbm.at[pl.ds(wid * b_per_w, b_per_w)])

    return pool


def _proj_body(w_ref, p_ref, o_ref):
    o_ref[...] = jax.lax.dot_general(
        w_ref[...],
        p_ref[...],
        (((0,), (0,)), ((), ())),
        preferred_element_type=jnp.float32,
    )


@functools.lru_cache(maxsize=None)
def _make_proj_kernel(B, K, V, BN):
    grid = pl.cdiv(V, BN)
    return pl.pallas_call(
        _proj_body,
        grid=(grid,),
        in_specs=[
            pl.BlockSpec((K, BN), lambda i: (0, i)),
            pl.BlockSpec((K, B), lambda i: (0, 0)),
        ],
        out_specs=pl.BlockSpec((BN, B), lambda i: (i, 0)),
        out_shape=jax.ShapeDtypeStruct((V, B), jnp.float32),
    )


def kernel(context, emb_table, W, b):
    B, CTX = context.shape
    V, D = emb_table.shape
    NC, NS = 2, 16
    idx_flat = jnp.clip(
        lax.optimization_barrier(context.reshape(-1)), 0, V - 1
    )
    pooled = _make_pool_kernel(B, CTX, D, NC, NS)(idx_flat, emb_table)
    w_aug = jnp.concatenate([W, b[None, :]], axis=0)
    p_aug = jnp.concatenate(
        [pooled.T, jnp.ones((1, B), jnp.float32)], axis=0
    )
    out_t = _make_proj_kernel(B, D + 1, V, 2048)(w_aug, p_aug)
    return out_t.T


# speedup vs baseline: 1.0520x; 1.0487x over previous
import functools

import jax
import jax.numpy as jnp
from jax import lax
from jax.experimental import pallas as pl
from jax.experimental.pallas import tpu as pltpu
from jax.experimental.pallas import tpu_sc as plsc

_LANES = 16


@functools.lru_cache(maxsize=None)
def _make_pool_kernel(B, CTX, D, NC, NS):
    NW = NC * NS
    b_per_w = B // NW
    n_idx = b_per_w * CTX
    inv_ctx = 1.0 / CTX

    mesh = plsc.VectorSubcoreMesh(
        core_axis_name="c", subcore_axis_name="s", num_cores=NC, num_subcores=NS
    )

    CH = 128
    n_chunks = n_idx // CH
    assert n_chunks * CH == n_idx

    @functools.partial(
        pl.kernel,
        mesh=mesh,
        out_type=jax.ShapeDtypeStruct((B, D), jnp.float32),
        scratch_types=[
            pltpu.VMEM((n_idx,), jnp.int32),
            pltpu.VMEM((n_idx, D), jnp.float32),
            pltpu.VMEM((b_per_w, D), jnp.float32),
            pltpu.SemaphoreType.DMA,
        ],
        compiler_params=pltpu.CompilerParams(use_tc_tiling_on_sc=False),
    )
    def pool(idx_hbm, table_hbm, out_hbm, idx_v, rows_v, pooled_v, sem):
        wid = lax.axis_index("s") * NC + lax.axis_index("c")
        pltpu.sync_copy(idx_hbm.at[pl.ds(wid * n_idx, n_idx)], idx_v)
        copies = [
            pltpu.async_copy(
                table_hbm.at[idx_v.at[pl.ds(c * CH, CH)]],
                rows_v.at[pl.ds(c * CH, CH)],
                sem,
            )
            for c in range(n_chunks)
        ]
        for cp in copies:
            cp.wait()

        def batch_body(i, carry):
            base = i * CTX

            def ctx_body(j, acc):
                a0, a1 = acc
                r = base + j
                a0 = a0 + rows_v[r, pl.ds(0, _LANES)]
                a1 = a1 + rows_v[r, pl.ds(_LANES, _LANES)]
                return (a0, a1)

            zero = jnp.zeros((_LANES,), jnp.float32)
            a0, a1 = lax.fori_loop(0, CTX, ctx_body, (zero, zero))
            pooled_v[i, pl.ds(0, _LANES)] = a0 * inv_ctx
            pooled_v[i, pl.ds(_LANES, _LANES)] = a1 * inv_ctx
            return carry

        lax.fori_loop(0, b_per_w, batch_body, 0)
        pltpu.sync_copy(pooled_v, out_hbm.at[pl.ds(wid * b_per_w, b_per_w)])

    return pool


def _proj_body(w_ref, p_ref, b_ref, o_ref):
    dn = (((0,), (0,)), ((), ()))
    ones = jnp.ones((1, p_ref.shape[1]), jnp.float32)
    o_ref[...] = jax.lax.dot_general(
        w_ref[...], p_ref[...], dn, preferred_element_type=jnp.float32
    ) + jax.lax.dot_general(
        b_ref[...], ones, dn, preferred_element_type=jnp.float32
    )


@functools.lru_cache(maxsize=None)
def _make_proj_kernel(B, K, V, BN):
    grid = pl.cdiv(V, BN)
    return pl.pallas_call(
        _proj_body,
        grid=(grid,),
        in_specs=[
            pl.BlockSpec((K, BN), lambda i: (0, i)),
            pl.BlockSpec((K, B), lambda i: (0, 0)),
            pl.BlockSpec((1, BN), lambda i: (0, i)),
        ],
        out_specs=pl.BlockSpec((BN, B), lambda i: (i, 0)),
        out_shape=jax.ShapeDtypeStruct((V, B), jnp.float32),
    )


def kernel(context, emb_table, W, b):
    B, CTX = context.shape
    V, D = emb_table.shape
    NC, NS = 2, 16
    idx_flat = jnp.clip(
        lax.optimization_barrier(context.reshape(-1)), 0, V - 1
    )
    pooled = _make_pool_kernel(B, CTX, D, NC, NS)(idx_flat, emb_table)
    out_t = _make_proj_kernel(B, D, V, 2048)(W, pooled.T, b.reshape(1, V))
    return out_t.T
